# Initial kernel scaffold; baseline (speedup 1.0000x reference)
import jax
import jax.numpy as jnp
from jax.experimental import pallas as pl


def kernel(buffer, observation_sequence, index, size):
    raise NotImplementedError("write your pallas kernel here")


# speedup vs baseline: 6.6082x; 6.6082x over previous
import functools

import jax
import jax.numpy as jnp
from jax import lax
from jax.experimental import pallas as pl
from jax.experimental.pallas import tpu as pltpu
from jax.experimental.pallas import tpu_sc as plsc

B = 64
CAP = 2048
SEQ = 1024
D = 256
NC, NS = 2, 16
NW = NC * NS
BPW = B // NW
CHUNK = 128
LANES = 16

_mesh = plsc.VectorSubcoreMesh(
    core_axis_name="c", subcore_axis_name="s", num_cores=NC, num_subcores=NS
)


@functools.partial(
    pl.kernel,
    out_type=jax.ShapeDtypeStruct((B * CAP, D), jnp.float32),
    mesh=_mesh,
    scratch_types=[
        pltpu.VMEM((B + LANES,), jnp.int32),
        pltpu.VMEM((CHUNK,), jnp.int32),
        pltpu.VMEM((CHUNK,), jnp.int32),
        pltpu.VMEM((CHUNK, D), jnp.float32),
        pltpu.VMEM((CHUNK, D), jnp.float32),
        pltpu.SemaphoreType.DMA,
        pltpu.SemaphoreType.DMA,
    ],
)
def _scatter(buf_hbm, obs_hbm, idx_hbm, out_hbm,
             idx_v, di_v, ui_v, da, db, sem_a, sem_b):
    wid = lax.axis_index("s") * NC + lax.axis_index("c")
    pltpu.sync_copy(idx_hbm, idx_v.at[pl.ds(0, B)])
    lanes = lax.iota(jnp.int32, LANES)

    for k in range(BPW):
        b = wid * BPW + k
        i = idx_v[pl.ds(b, LANES)][0]
        out_base = b * CAP

        def fill_ids(c, start, idx_ref):
            for v in range(CHUNK // LANES):
                off = start + c * CHUNK + v * LANES + lanes
                idx_ref[pl.ds(v * LANES, LANES)] = out_base + (off & (CAP - 1))

        def obs_chunk(c, carry):
            fill_ids(c, i, di_v)
            pltpu.sync_copy(obs_hbm.at[pl.ds(b * SEQ + c * CHUNK, CHUNK)], da)
            pltpu.async_copy(da, out_hbm.at[di_v], sem_a).wait()
            return carry

        def buf_chunk(c, carry):
            fill_ids(c, i + SEQ, ui_v)
            pltpu.async_copy(buf_hbm.at[ui_v], db, sem_b).wait()
            pltpu.async_copy(db, out_hbm.at[ui_v], sem_b).wait()
            return carry

        lax.fori_loop(0, SEQ // CHUNK, obs_chunk, 0)
        lax.fori_loop(0, SEQ // CHUNK, buf_chunk, 0)


def kernel(buffer, observation_sequence, index, size):
    del size
    buf2d = buffer.reshape(B * CAP, D)
    obs2d = observation_sequence.reshape(B * SEQ, D)
    out2d = _scatter(buf2d, obs2d, index)
    return out2d.reshape(B, CAP, D)


# speedup vs baseline: 7.3214x; 1.1079x over previous
import functools

import jax
import jax.numpy as jnp
from jax import lax
from jax.experimental import pallas as pl
from jax.experimental.pallas import tpu as pltpu
from jax.experimental.pallas import tpu_sc as plsc

B = 64
CAP = 2048
SEQ = 1024
D = 256
NC, NS = 2, 16
NW = NC * NS
BPW = B // NW
CHUNK = 128
LANES = 16
DEPTH = 3

_mesh = plsc.VectorSubcoreMesh(
    core_axis_name="c", subcore_axis_name="s", num_cores=NC, num_subcores=NS
)


@functools.partial(
    pl.kernel,
    out_type=jax.ShapeDtypeStruct((B * CAP, D), jnp.float32),
    mesh=_mesh,
    scratch_types=(
        [pltpu.VMEM((B + LANES,), jnp.int32)]
        + [pltpu.VMEM((CHUNK,), jnp.int32) for _ in range(DEPTH)]
        + [pltpu.VMEM((CHUNK, D), jnp.float32) for _ in range(DEPTH)]
        + [pltpu.SemaphoreType.DMA for _ in range(2 * DEPTH)]
    ),
)
def _scatter(buf_hbm, obs_hbm, idx_hbm, out_hbm, idx_v, *scratch):
    idx_refs = scratch[:DEPTH]
    data_refs = scratch[DEPTH:2 * DEPTH]
    sems_in = scratch[2 * DEPTH:3 * DEPTH]
    sems_out = scratch[3 * DEPTH:4 * DEPTH]

    wid = lax.axis_index("s") * NC + lax.axis_index("c")
    pltpu.sync_copy(idx_hbm, idx_v.at[pl.ds(0, B)])
    lanes = lax.iota(jnp.int32, LANES)

    starts = []
    for k in range(BPW):
        b = wid * BPW + k
        starts.append(idx_v[pl.ds(b, LANES)][0])

    jobs = []
    for k in range(BPW):
        b = wid * BPW + k
        i = starts[k]
        for c in range(SEQ // CHUNK):
            jobs.append(("obs", b, i, c))
        for c in range(SEQ // CHUNK):
            jobs.append(("buf", b, i, c))

    descs_out = [None] * len(jobs)
    descs_in = [None] * len(jobs)

    for j, (kind, b, i, c) in enumerate(jobs):
        s = j % DEPTH
        if j >= DEPTH:
            descs_out[j - DEPTH].wait()
        circ0 = i + c * CHUNK + (SEQ if kind == "buf" else 0)
        for v in range(CHUNK // LANES):
            off = circ0 + v * LANES + lanes
            idx_refs[s][pl.ds(v * LANES, LANES)] = b * CAP + (off & (CAP - 1))
        if kind == "obs":
            descs_in[j] = pltpu.async_copy(
                obs_hbm.at[pl.ds(b * SEQ + c * CHUNK, CHUNK)],
                data_refs[s], sems_in[s])
        else:
            descs_in[j] = pltpu.async_copy(
                buf_hbm.at[idx_refs[s]], data_refs[s], sems_in[s])
        if j >= 1:
            sp = (j - 1) % DEPTH
            descs_in[j - 1].wait()
            descs_out[j - 1] = pltpu.async_copy(
                data_refs[sp], out_hbm.at[idx_refs[sp]], sems_out[sp])

    j_last = len(jobs) - 1
    descs_in[j_last].wait()
    descs_out[j_last] = pltpu.async_copy(
        data_refs[j_last % DEPTH], out_hbm.at[idx_refs[j_last % DEPTH]],
        sems_out[j_last % DEPTH])
    for j in range(max(0, len(jobs) - DEPTH), len(jobs)):
        descs_out[j].wait()


def kernel(buffer, observation_sequence, index, size):
    del size
    buf2d = buffer.reshape(B * CAP, D)
    obs2d = observation_sequence.reshape(B * SEQ, D)
    out2d = _scatter(buf2d, obs2d, index)
    return out2d.reshape(B, CAP, D)
